# baseline (speedup 1.0000x reference)
import jax
import jax.numpy as jnp
from jax.experimental import pallas as pl
from jax.experimental.pallas import tpu as pltpu

TOKENS = 8192
D_MODEL = 4096
UNITS = 64
TEMP = 0.5

BLOCK_M = 512
TILES = TOKENS // BLOCK_M
NBUF = 4


def _router_kernel(x_hbm, w_ref, b_ref, o_ref, buf, sems):
    def copy(i):
        return pltpu.make_async_copy(
            x_hbm.at[pl.ds(i * BLOCK_M, BLOCK_M), :],
            buf.at[i % NBUF],
            sems.at[i % NBUF],
        )

    for i in range(NBUF):
        copy(i).start()
    for i in range(TILES):
        copy(i).wait()
        logits = jnp.dot(buf[i % NBUF], w_ref[...],
                         preferred_element_type=jnp.float32)
        logits = logits + b_ref[...]
        o_ref[pl.ds(i * BLOCK_M, BLOCK_M), :] = jax.nn.sigmoid(
            logits / (TEMP + 1e-08))
        if i + NBUF < TILES:
            copy(i + NBUF).start()


def kernel(inputs, proj, logit_bias):
    bias2d = logit_bias.reshape(1, UNITS)
    return pl.pallas_call(
        _router_kernel,
        in_specs=[
            pl.BlockSpec(memory_space=pltpu.MemorySpace.HBM),
            pl.BlockSpec(memory_space=pltpu.MemorySpace.VMEM),
            pl.BlockSpec(memory_space=pltpu.MemorySpace.VMEM),
        ],
        out_specs=pl.BlockSpec(memory_space=pltpu.MemorySpace.VMEM),
        out_shape=jax.ShapeDtypeStruct((TOKENS, UNITS), jnp.float32),
        scratch_shapes=[
            pltpu.VMEM((NBUF, BLOCK_M, D_MODEL), jnp.float32),
            pltpu.SemaphoreType.DMA((NBUF,)),
        ],
        compiler_params=pltpu.CompilerParams(
            vmem_limit_bytes=100 * 1024 * 1024,
        ),
    )(inputs, proj, bias2d)
